# baseline (speedup 1.0000x reference)
import functools
import jax
import jax.numpy as jnp
from jax import lax
from jax.experimental import pallas as pl
from jax.experimental.pallas import tpu as pltpu
from jax.experimental.pallas import tpu_sc as plsc


CHUNK_ROWS = 32


def kernel(tokens, W_pos):
    batch = tokens.shape[0]
    seq_len = tokens.shape[1]
    d = W_pos.shape[1]

    info = plsc.get_sparse_core_info()
    nw = info.num_cores * info.num_subcores
    rows_per_w = seq_len // nw
    nchunk = rows_per_w // CHUNK_ROWS

    mesh = plsc.VectorSubcoreMesh(core_axis_name="c", subcore_axis_name="s")

    @functools.partial(
        pl.kernel,
        mesh=mesh,
        out_type=jax.ShapeDtypeStruct((batch, seq_len, d), W_pos.dtype),
        scratch_types=[pltpu.VMEM((CHUNK_ROWS, d), W_pos.dtype)],
    )
    def sc_copy(w_hbm, out_hbm, buf):
        wid = lax.axis_index("s") * info.num_cores + lax.axis_index("c")
        base = wid * rows_per_w
        for i in range(nchunk):
            r = base + i * CHUNK_ROWS
            pltpu.sync_copy(w_hbm.at[pl.ds(r, CHUNK_ROWS), :], buf)
            for b in range(batch):
                pltpu.sync_copy(buf, out_hbm.at[b, pl.ds(r, CHUNK_ROWS), :])

    return sc_copy(W_pos[:seq_len])
